# baseline (speedup 1.0000x reference)
import functools

import jax
import jax.numpy as jnp
from jax import lax
from jax.experimental import pallas as pl
from jax.experimental.pallas import tpu as pltpu
from jax.experimental.pallas import tpu_sc as plsc

N = 10000
E = 320000
D = 128
K = 16

NC = 2
NS = 16
NW = NC * NS
EW = E // NW
G = 128
NB = 80
NBH = NB // 2
EWP = NB * G
RPS = N // NS
DUMP = N
DEGP = 10240
BN = 1000
NBLK = N // BN

NH = N // 2
AGH = NH + 1
G2 = 64
ET = E // NS
NBT = 320
CH = 8
NCHK = NBT // CH
TRW = 312
YRW = 624


def _mesh():
    return plsc.VectorSubcoreMesh(
        core_axis_name="c", subcore_axis_name="s",
        num_cores=NC, num_subcores=NS)


def _deg_body(rowd_hbm, out_hbm, idx_v, ones_v, zbuf_v, deg_sh):
    c = lax.axis_index("c")
    s = lax.axis_index("s")
    wid = c * NS + s
    pltpu.sync_copy(rowd_hbm.at[wid], idx_v)
    for i in range(G // 16):
        ones_v[pl.ds(i * 16, 16)] = jnp.ones((16,), jnp.float32)

    @pl.when(s == 0)
    def _zero():
        def zfill(i, carry):
            zbuf_v[pl.ds(i * 16, 16)] = jnp.zeros((16,), jnp.float32)
            return carry
        lax.fori_loop(0, 128, zfill, 0)
        for t in range(5):
            pltpu.sync_copy(zbuf_v, deg_sh.at[pl.ds(t * 2048, 2048)])

    plsc.subcore_barrier()

    def body(j, carry):
        pltpu.sync_copy(ones_v, deg_sh.at[idx_v.at[j]], add=True)
        return carry
    lax.fori_loop(0, NB, body, 0)

    plsc.subcore_barrier()

    @pl.when(s == 0)
    def _out():
        pltpu.sync_copy(deg_sh, out_hbm.at[c])


def _deg_call(rowd):
    f = pl.kernel(
        _deg_body,
        out_type=jax.ShapeDtypeStruct((NC, DEGP), jnp.float32),
        mesh=_mesh(),
        scratch_types=[
            pltpu.VMEM((NB, G), jnp.int32),
            pltpu.VMEM((G,), jnp.float32),
            pltpu.VMEM((2048,), jnp.float32),
            pltpu.VMEM_SHARED((DEGP,), jnp.float32),
        ],
    )
    return f(rowd)


def _spmv_body(y_hbm, rowg_hbm, colg_hbm, out_hbm,
               idxr_v, idxc_v, rows_v, y_sh, agg_sh, semg):
    c = lax.axis_index("c")
    s = lax.axis_index("s")

    pltpu.sync_copy(y_hbm.at[pl.ds(s * YRW, YRW)], y_sh.at[pl.ds(s * YRW, YRW)])

    @pl.when(s == NS - 1)
    def _ytail():
        pltpu.sync_copy(y_hbm.at[pl.ds(NS * YRW, N - NS * YRW)],
                        y_sh.at[pl.ds(NS * YRW, N - NS * YRW)])

    def zfill(i, carry):
        for jj in range(8):
            rows_v[i, pl.ds(jj * 16, 16)] = jnp.zeros((16,), jnp.float32)
        return carry
    lax.fori_loop(0, G2, zfill, 0)
    for t in range(TRW // G2):
        pltpu.sync_copy(rows_v, agg_sh.at[pl.ds(s * TRW + t * G2, G2)])
    pltpu.sync_copy(rows_v.at[pl.ds(0, TRW % G2)],
                    agg_sh.at[pl.ds(s * TRW + (TRW // G2) * G2, TRW % G2)])

    @pl.when(s == NS - 1)
    def _ztail():
        pltpu.sync_copy(rows_v.at[pl.ds(0, AGH - NS * TRW)],
                        agg_sh.at[pl.ds(NS * TRW, AGH - NS * TRW)])

    plsc.subcore_barrier()

    def chunk(ch, carry):
        pltpu.sync_copy(rowg_hbm.at[s, pl.ds(ch * CH, CH)], idxr_v)
        pltpu.sync_copy(colg_hbm.at[c, s, pl.ds(ch * CH, CH)], idxc_v)
        for b in range(CH):
            pltpu.async_copy(y_sh.at[idxr_v.at[b]], rows_v, semg).wait()
            pltpu.sync_copy(rows_v, agg_sh.at[idxc_v.at[b]], add=True)
        return carry
    lax.fori_loop(0, NCHK, chunk, 0)

    plsc.subcore_barrier()
    pltpu.sync_copy(agg_sh.at[pl.ds(s * TRW, TRW)],
                    out_hbm.at[pl.ds(c * NH + s * TRW, TRW)])

    @pl.when(s == NS - 1)
    def _tail():
        pltpu.sync_copy(agg_sh.at[pl.ds(NS * TRW, NH - NS * TRW)],
                        out_hbm.at[pl.ds(c * NH + NS * TRW, NH - NS * TRW)])


def _spmv_call(y, rowg, colg):
    f = pl.kernel(
        _spmv_body,
        out_type=jax.ShapeDtypeStruct((N, D), jnp.float32),
        mesh=_mesh(),
        scratch_types=[
            pltpu.VMEM((CH, G2), jnp.int32),
            pltpu.VMEM((CH, G2), jnp.int32),
            pltpu.VMEM((G2, D), jnp.float32),
            pltpu.VMEM_SHARED((N, D), jnp.float32),
            pltpu.VMEM_SHARED((AGH, D), jnp.float32),
            pltpu.SemaphoreType.DMA,
        ],
    )
    return f(y, rowg, colg)


def _prep_body(degp_ref, x_ref, dis_ref, diag_ref, y_ref):
    deg = degp_ref[0] + degp_ref[1]
    pos = deg > 0.0
    dis = jnp.where(pos, lax.rsqrt(jnp.where(pos, deg, 1.0)), 0.0)
    dis_ref[...] = dis
    diag_ref[...] = jnp.where(pos, 0.0, -1.0)
    y_ref[...] = dis * x_ref[...]


def _prep_call(degp, x):
    return pl.pallas_call(
        _prep_body,
        grid=(NBLK,),
        in_specs=[
            pl.BlockSpec((NC, BN, 1), lambda i: (0, i, 0)),
            pl.BlockSpec((BN, D), lambda i: (i, 0)),
        ],
        out_specs=[
            pl.BlockSpec((BN, 1), lambda i: (i, 0)),
            pl.BlockSpec((BN, 1), lambda i: (i, 0)),
            pl.BlockSpec((BN, D), lambda i: (i, 0)),
        ],
        out_shape=[
            jax.ShapeDtypeStruct((N, 1), jnp.float32),
            jax.ShapeDtypeStruct((N, 1), jnp.float32),
            jax.ShapeDtypeStruct((N, D), jnp.float32),
        ],
    )(degp, x)


def _update_body(agg_ref, xc_ref, xo_ref, dis_ref, diag_ref, tx_ref, y_ref,
                 *, alpha, beta):
    dis = dis_ref[...]
    lap = diag_ref[...] * xc_ref[...] - dis * agg_ref[...]
    txn = alpha * lap - beta * xo_ref[...]
    tx_ref[...] = txn
    y_ref[...] = dis * txn


def _update_call(agg, xc, xo, dis, diag, alpha, beta):
    return pl.pallas_call(
        functools.partial(_update_body, alpha=alpha, beta=beta),
        grid=(NBLK,),
        in_specs=[
            pl.BlockSpec((BN, D), lambda i: (i, 0)),
            pl.BlockSpec((BN, D), lambda i: (i, 0)),
            pl.BlockSpec((BN, D), lambda i: (i, 0)),
            pl.BlockSpec((BN, 1), lambda i: (i, 0)),
            pl.BlockSpec((BN, 1), lambda i: (i, 0)),
        ],
        out_specs=[
            pl.BlockSpec((BN, D), lambda i: (i, 0)),
            pl.BlockSpec((BN, D), lambda i: (i, 0)),
        ],
        out_shape=[
            jax.ShapeDtypeStruct((N, D), jnp.float32),
            jax.ShapeDtypeStruct((N, D), jnp.float32),
        ],
    )(agg, xc, xo, dis, diag)


def _mm_body(tx_ref, w_ref, b_ref, out_ref, acc_ref):
    k = pl.program_id(1)

    @pl.when(k == 0)
    def _init():
        acc_ref[...] = jnp.zeros_like(acc_ref)

    acc_ref[...] += jnp.dot(tx_ref[0], w_ref[0],
                            preferred_element_type=jnp.float32)

    @pl.when(k == K - 1)
    def _fin():
        out_ref[...] = jnp.maximum(acc_ref[...] + b_ref[...], 0.0)


def _mm_call(txstack, W, b2):
    return pl.pallas_call(
        _mm_body,
        grid=(NBLK, K),
        in_specs=[
            pl.BlockSpec((1, BN, D), lambda i, k: (k, i, 0)),
            pl.BlockSpec((1, D, D), lambda i, k: (k, 0, 0)),
            pl.BlockSpec((1, D), lambda i, k: (0, 0)),
        ],
        out_specs=pl.BlockSpec((BN, D), lambda i, k: (i, 0)),
        out_shape=jax.ShapeDtypeStruct((N, D), jnp.float32),
        scratch_shapes=[pltpu.VMEM((BN, D), jnp.float32)],
        compiler_params=pltpu.CompilerParams(
            dimension_semantics=("parallel", "arbitrary")),
    )(txstack, W, b2)


def kernel(node_emb, edge_index, W, b):
    row = edge_index[0]
    col = edge_index[1]
    row_w = row.reshape(NW, EW)
    row_t = row.reshape(NS, ET)
    col_t = col.reshape(NS, ET)
    rowg = jnp.pad(row_t, ((0, 0), (0, NBT * G2 - ET))).reshape(NS, NBT, G2)
    colh = []
    for cidx in range(NC):
        lo = cidx * NH
        inh = (col_t >= lo) & (col_t < lo + NH)
        cc = jnp.where(inh, col_t - lo, NH)
        colh.append(jnp.pad(cc, ((0, 0), (0, NBT * G2 - ET)),
                            constant_values=NH))
    colg = jnp.stack(colh).reshape(NC, NS, NBT, G2)
    rowd = jnp.pad(row_w, ((0, 0), (0, EWP - EW)),
                   constant_values=DUMP).reshape(NW, NB, G)

    degp = _deg_call(rowd)
    dis, diag, y = _prep_call(degp[:, :N].reshape(NC, N, 1), node_emb)

    txs = [node_emb]
    agg = _spmv_call(y, rowg, colg)
    tx, y = _update_call(agg, node_emb, node_emb, dis, diag, 1.0, 0.0)
    txs.append(tx)
    xo, xc = node_emb, tx
    for _ in range(2, K):
        agg = _spmv_call(y, rowg, colg)
        tx, y = _update_call(agg, xc, xo, dis, diag, 2.0, 1.0)
        txs.append(tx)
        xo, xc = xc, tx

    txstack = jnp.stack(txs, axis=0)
    return _mm_call(txstack, W, b.reshape(1, D))


# speedup vs baseline: 1.2446x; 1.2446x over previous
import functools

import jax
import jax.numpy as jnp
from jax import lax
from jax.experimental import pallas as pl
from jax.experimental.pallas import tpu as pltpu
from jax.experimental.pallas import tpu_sc as plsc

N = 10000
E = 320000
D = 128
K = 16

NC = 2
NS = 16
NW = NC * NS
EW = E // NW
G = 128
NB = 80
NBH = NB // 2
EWP = NB * G
RPS = N // NS
DUMP = N
DEGP = 10240
BN = 1000
NBLK = N // BN

NH = N // 2
AGH = NH + 1
G2 = 64
ET = E // NS
NBT = 320
CH = 8
NCHK = NBT // CH
TRW = 312
YRW = 624
CAP = 5888
NBT2 = 2 * CAP // G2
NCHK2 = NBT2 // CH


def _mesh():
    return plsc.VectorSubcoreMesh(
        core_axis_name="c", subcore_axis_name="s",
        num_cores=NC, num_subcores=NS)


def _deg_body(rowd_hbm, out_hbm, idx_v, ones_v, zbuf_v, deg_sh):
    c = lax.axis_index("c")
    s = lax.axis_index("s")
    wid = c * NS + s
    pltpu.sync_copy(rowd_hbm.at[wid], idx_v)
    for i in range(G // 16):
        ones_v[pl.ds(i * 16, 16)] = jnp.ones((16,), jnp.float32)

    @pl.when(s == 0)
    def _zero():
        def zfill(i, carry):
            zbuf_v[pl.ds(i * 16, 16)] = jnp.zeros((16,), jnp.float32)
            return carry
        lax.fori_loop(0, 128, zfill, 0)
        for t in range(5):
            pltpu.sync_copy(zbuf_v, deg_sh.at[pl.ds(t * 2048, 2048)])

    plsc.subcore_barrier()

    def body(j, carry):
        pltpu.sync_copy(ones_v, deg_sh.at[idx_v.at[j]], add=True)
        return carry
    lax.fori_loop(0, NB, body, 0)

    plsc.subcore_barrier()

    @pl.when(s == 0)
    def _out():
        pltpu.sync_copy(deg_sh, out_hbm.at[c])


def _deg_call(rowd):
    f = pl.kernel(
        _deg_body,
        out_type=jax.ShapeDtypeStruct((NC, DEGP), jnp.float32),
        mesh=_mesh(),
        scratch_types=[
            pltpu.VMEM((NB, G), jnp.int32),
            pltpu.VMEM((G,), jnp.float32),
            pltpu.VMEM((2048,), jnp.float32),
            pltpu.VMEM_SHARED((DEGP,), jnp.float32),
        ],
    )
    return f(rowd)


def _pos_body(col_ref, l128_ref, s80_ref, one_ref, pos_ref, colloc_ref):
    w = pl.program_id(0)
    colv = col_ref[0]
    ism = colv < NH
    m = ism.astype(jnp.float32)
    m1 = 1.0 - m
    a0 = jnp.dot(m, l128_ref[...], preferred_element_type=jnp.float32)
    a1 = jnp.dot(m1, l128_ref[...], preferred_element_type=jnp.float32)
    rs0 = jnp.dot(m, one_ref[...], preferred_element_type=jnp.float32)
    rs1 = jnp.dot(m1, one_ref[...], preferred_element_type=jnp.float32)
    b0 = jnp.dot(s80_ref[...], rs0, preferred_element_type=jnp.float32)
    b1 = jnp.dot(s80_ref[...], rs1, preferred_element_type=jnp.float32)
    base0 = (w * CAP - 1).astype(jnp.float32)
    base1 = (NW * CAP + w * CAP - 1).astype(jnp.float32)
    p0 = a0 + b0 + base0
    p1 = a1 + b1 + base1
    pos_ref[0] = jnp.where(ism, p0, p1).astype(jnp.int32)
    colloc_ref[0] = jnp.where(ism, colv, colv - NH)


def _pos_call(colpart):
    ar = jnp.arange(G, dtype=jnp.int32)
    l128 = (ar[:, None] <= ar[None, :]).astype(jnp.float32)
    ag = jnp.arange(NB, dtype=jnp.int32)
    s80 = (ag[:, None] > ag[None, :]).astype(jnp.float32)
    one = jnp.ones((G, 1), jnp.float32)
    return pl.pallas_call(
        _pos_body,
        grid=(NW,),
        in_specs=[
            pl.BlockSpec((1, NB, G), lambda w: (w, 0, 0)),
            pl.BlockSpec((G, G), lambda w: (0, 0)),
            pl.BlockSpec((NB, NB), lambda w: (0, 0)),
            pl.BlockSpec((G, 1), lambda w: (0, 0)),
        ],
        out_specs=[
            pl.BlockSpec((1, NB, G), lambda w: (w, 0, 0)),
            pl.BlockSpec((1, NB, G), lambda w: (w, 0, 0)),
        ],
        out_shape=[
            jax.ShapeDtypeStruct((NW, NB, G), jnp.int32),
            jax.ShapeDtypeStruct((NW, NB, G), jnp.int32),
        ],
    )(colpart, l128, s80, one)


def _part_body(row_hbm, colloc_hbm, pos_hbm, rowp_hbm, colp_hbm,
               row_v, col_v, pos_v, pad_v):
    c = lax.axis_index("c")
    s = lax.axis_index("s")
    w = c * NS + s

    def fillz(i, carry):
        pad_v[pl.ds(i * 16, 16)] = jnp.zeros((16,), jnp.int32)
        return carry
    lax.fori_loop(0, 128, fillz, 0)
    for h in range(2):
        base = h * NW * CAP + w * CAP
        for off, sz in ((0, 2048), (2048, 2048), (4096, CAP - 4096)):
            pltpu.sync_copy(pad_v.at[pl.ds(0, sz)],
                            rowp_hbm.at[pl.ds(base + off, sz)])

    def filld(i, carry):
        pad_v[pl.ds(i * 16, 16)] = jnp.full((16,), NH, jnp.int32)
        return carry
    lax.fori_loop(0, 128, filld, 0)
    for h in range(2):
        base = h * NW * CAP + w * CAP
        for off, sz in ((0, 2048), (2048, 2048), (4096, CAP - 4096)):
            pltpu.sync_copy(pad_v.at[pl.ds(0, sz)],
                            colp_hbm.at[pl.ds(base + off, sz)])

    pltpu.sync_copy(row_hbm.at[w], row_v)
    pltpu.sync_copy(colloc_hbm.at[w], col_v)
    pltpu.sync_copy(pos_hbm.at[w], pos_v)

    def body(j, carry):
        pltpu.sync_copy(row_v.at[j], rowp_hbm.at[pos_v.at[j]])
        pltpu.sync_copy(col_v.at[j], colp_hbm.at[pos_v.at[j]])
        return carry
    lax.fori_loop(0, NB, body, 0)


def _part_call(rowpart, colloc, pos):
    f = pl.kernel(
        _part_body,
        out_type=(jax.ShapeDtypeStruct((NC * NW * CAP,), jnp.int32),
                  jax.ShapeDtypeStruct((NC * NW * CAP,), jnp.int32)),
        mesh=_mesh(),
        scratch_types=[
            pltpu.VMEM((NB, G), jnp.int32),
            pltpu.VMEM((NB, G), jnp.int32),
            pltpu.VMEM((NB, G), jnp.int32),
            pltpu.VMEM((2048,), jnp.int32),
        ],
    )
    return f(rowpart, colloc, pos)


def _spmv_body(y_hbm, rowg_hbm, colg_hbm, out_hbm,
               idxr_v, idxc_v, rows_v, y_sh, agg_sh, semg):
    c = lax.axis_index("c")
    s = lax.axis_index("s")

    pltpu.sync_copy(y_hbm.at[pl.ds(s * YRW, YRW)], y_sh.at[pl.ds(s * YRW, YRW)])

    @pl.when(s == NS - 1)
    def _ytail():
        pltpu.sync_copy(y_hbm.at[pl.ds(NS * YRW, N - NS * YRW)],
                        y_sh.at[pl.ds(NS * YRW, N - NS * YRW)])

    def zfill(i, carry):
        for jj in range(8):
            rows_v[i, pl.ds(jj * 16, 16)] = jnp.zeros((16,), jnp.float32)
        return carry
    lax.fori_loop(0, G2, zfill, 0)
    for t in range(TRW // G2):
        pltpu.sync_copy(rows_v, agg_sh.at[pl.ds(s * TRW + t * G2, G2)])
    pltpu.sync_copy(rows_v.at[pl.ds(0, TRW % G2)],
                    agg_sh.at[pl.ds(s * TRW + (TRW // G2) * G2, TRW % G2)])

    @pl.when(s == NS - 1)
    def _ztail():
        pltpu.sync_copy(rows_v.at[pl.ds(0, AGH - NS * TRW)],
                        agg_sh.at[pl.ds(NS * TRW, AGH - NS * TRW)])

    plsc.subcore_barrier()

    def chunk(ch, carry):
        pltpu.sync_copy(rowg_hbm.at[c, s, pl.ds(ch * CH, CH)], idxr_v)
        pltpu.sync_copy(colg_hbm.at[c, s, pl.ds(ch * CH, CH)], idxc_v)
        for b in range(CH):
            pltpu.async_copy(y_sh.at[idxr_v.at[b]], rows_v, semg).wait()
            pltpu.sync_copy(rows_v, agg_sh.at[idxc_v.at[b]], add=True)
        return carry
    lax.fori_loop(0, NCHK2, chunk, 0)

    plsc.subcore_barrier()
    pltpu.sync_copy(agg_sh.at[pl.ds(s * TRW, TRW)],
                    out_hbm.at[pl.ds(c * NH + s * TRW, TRW)])

    @pl.when(s == NS - 1)
    def _tail():
        pltpu.sync_copy(agg_sh.at[pl.ds(NS * TRW, NH - NS * TRW)],
                        out_hbm.at[pl.ds(c * NH + NS * TRW, NH - NS * TRW)])


def _spmv_call(y, rowg, colg):
    f = pl.kernel(
        _spmv_body,
        out_type=jax.ShapeDtypeStruct((N, D), jnp.float32),
        mesh=_mesh(),
        scratch_types=[
            pltpu.VMEM((CH, G2), jnp.int32),
            pltpu.VMEM((CH, G2), jnp.int32),
            pltpu.VMEM((G2, D), jnp.float32),
            pltpu.VMEM_SHARED((N, D), jnp.float32),
            pltpu.VMEM_SHARED((AGH, D), jnp.float32),
            pltpu.SemaphoreType.DMA,
        ],
    )
    return f(y, rowg, colg)


def _prep_body(degp_ref, x_ref, dis_ref, diag_ref, y_ref):
    deg = degp_ref[0] + degp_ref[1]
    pos = deg > 0.0
    dis = jnp.where(pos, lax.rsqrt(jnp.where(pos, deg, 1.0)), 0.0)
    dis_ref[...] = dis
    diag_ref[...] = jnp.where(pos, 0.0, -1.0)
    y_ref[...] = dis * x_ref[...]


def _prep_call(degp, x):
    return pl.pallas_call(
        _prep_body,
        grid=(NBLK,),
        in_specs=[
            pl.BlockSpec((NC, BN, 1), lambda i: (0, i, 0)),
            pl.BlockSpec((BN, D), lambda i: (i, 0)),
        ],
        out_specs=[
            pl.BlockSpec((BN, 1), lambda i: (i, 0)),
            pl.BlockSpec((BN, 1), lambda i: (i, 0)),
            pl.BlockSpec((BN, D), lambda i: (i, 0)),
        ],
        out_shape=[
            jax.ShapeDtypeStruct((N, 1), jnp.float32),
            jax.ShapeDtypeStruct((N, 1), jnp.float32),
            jax.ShapeDtypeStruct((N, D), jnp.float32),
        ],
    )(degp, x)


def _update_body(agg_ref, xc_ref, xo_ref, dis_ref, diag_ref, tx_ref, y_ref,
                 *, alpha, beta):
    dis = dis_ref[...]
    lap = diag_ref[...] * xc_ref[...] - dis * agg_ref[...]
    txn = alpha * lap - beta * xo_ref[...]
    tx_ref[...] = txn
    y_ref[...] = dis * txn


def _update_call(agg, xc, xo, dis, diag, alpha, beta):
    return pl.pallas_call(
        functools.partial(_update_body, alpha=alpha, beta=beta),
        grid=(NBLK,),
        in_specs=[
            pl.BlockSpec((BN, D), lambda i: (i, 0)),
            pl.BlockSpec((BN, D), lambda i: (i, 0)),
            pl.BlockSpec((BN, D), lambda i: (i, 0)),
            pl.BlockSpec((BN, 1), lambda i: (i, 0)),
            pl.BlockSpec((BN, 1), lambda i: (i, 0)),
        ],
        out_specs=[
            pl.BlockSpec((BN, D), lambda i: (i, 0)),
            pl.BlockSpec((BN, D), lambda i: (i, 0)),
        ],
        out_shape=[
            jax.ShapeDtypeStruct((N, D), jnp.float32),
            jax.ShapeDtypeStruct((N, D), jnp.float32),
        ],
    )(agg, xc, xo, dis, diag)


def _mm_body(tx_ref, w_ref, b_ref, out_ref, acc_ref):
    k = pl.program_id(1)

    @pl.when(k == 0)
    def _init():
        acc_ref[...] = jnp.zeros_like(acc_ref)

    acc_ref[...] += jnp.dot(tx_ref[0], w_ref[0],
                            preferred_element_type=jnp.float32)

    @pl.when(k == K - 1)
    def _fin():
        out_ref[...] = jnp.maximum(acc_ref[...] + b_ref[...], 0.0)


def _mm_call(txstack, W, b2):
    return pl.pallas_call(
        _mm_body,
        grid=(NBLK, K),
        in_specs=[
            pl.BlockSpec((1, BN, D), lambda i, k: (k, i, 0)),
            pl.BlockSpec((1, D, D), lambda i, k: (k, 0, 0)),
            pl.BlockSpec((1, D), lambda i, k: (0, 0)),
        ],
        out_specs=pl.BlockSpec((BN, D), lambda i, k: (i, 0)),
        out_shape=jax.ShapeDtypeStruct((N, D), jnp.float32),
        scratch_shapes=[pltpu.VMEM((BN, D), jnp.float32)],
        compiler_params=pltpu.CompilerParams(
            dimension_semantics=("parallel", "arbitrary")),
    )(txstack, W, b2)


def kernel(node_emb, edge_index, W, b):
    row = edge_index[0]
    col = edge_index[1]
    row_w = row.reshape(NW, EW)
    col_w = col.reshape(NW, EW)
    rowpart = jnp.pad(row_w, ((0, 0), (0, EWP - EW))).reshape(NW, NB, G)
    colpart = jnp.pad(col_w, ((0, 0), (0, EWP - EW)),
                      constant_values=N).reshape(NW, NB, G)
    rowd = jnp.pad(row_w, ((0, 0), (0, EWP - EW)),
                   constant_values=DUMP).reshape(NW, NB, G)

    pos, colloc = _pos_call(colpart)
    rowp, colp = _part_call(rowpart, colloc, pos)
    rowg = rowp.reshape(NC, NS, NBT2, G2)
    colg = colp.reshape(NC, NS, NBT2, G2)

    degp = _deg_call(rowd)
    dis, diag, y = _prep_call(degp[:, :N].reshape(NC, N, 1), node_emb)

    txs = [node_emb]
    agg = _spmv_call(y, rowg, colg)
    tx, y = _update_call(agg, node_emb, node_emb, dis, diag, 1.0, 0.0)
    txs.append(tx)
    xo, xc = node_emb, tx
    for _ in range(2, K):
        agg = _spmv_call(y, rowg, colg)
        tx, y = _update_call(agg, xc, xo, dis, diag, 2.0, 1.0)
        txs.append(tx)
        xo, xc = xc, tx

    txstack = jnp.stack(txs, axis=0)
    return _mm_call(txstack, W, b.reshape(1, D))


# speedup vs baseline: 1.6404x; 1.3180x over previous
import functools

import jax
import jax.numpy as jnp
from jax import lax
from jax.experimental import pallas as pl
from jax.experimental.pallas import tpu as pltpu
from jax.experimental.pallas import tpu_sc as plsc

N = 10000
E = 320000
D = 128
K = 16

NC = 2
NS = 16
NW = NC * NS
EW = E // NW
G = 128
NB = 80
NBH = NB // 2
EWP = NB * G
RPS = N // NS
DUMP = N
DEGP = 10240
BN = 1000
NBLK = N // BN

NH = N // 2
AGH = NH + 1
G2 = 64
ET = E // NS
NBT = 320
CH = 8
NCHK = NBT // CH
TRW = 312
YRW = 624
CAP = 5888
NBT2 = 2 * CAP // G2
NCHK2 = NBT2 // CH


def _mesh():
    return plsc.VectorSubcoreMesh(
        core_axis_name="c", subcore_axis_name="s",
        num_cores=NC, num_subcores=NS)


def _deg_body(rowd_hbm, out_hbm, idx_v, ones_v, zbuf_v, deg_sh):
    c = lax.axis_index("c")
    s = lax.axis_index("s")
    wid = c * NS + s
    pltpu.sync_copy(rowd_hbm.at[wid], idx_v)
    for i in range(G // 16):
        ones_v[pl.ds(i * 16, 16)] = jnp.ones((16,), jnp.float32)

    @pl.when(s == 0)
    def _zero():
        def zfill(i, carry):
            zbuf_v[pl.ds(i * 16, 16)] = jnp.zeros((16,), jnp.float32)
            return carry
        lax.fori_loop(0, 128, zfill, 0)
        for t in range(5):
            pltpu.sync_copy(zbuf_v, deg_sh.at[pl.ds(t * 2048, 2048)])

    plsc.subcore_barrier()

    def body(j, carry):
        pltpu.sync_copy(ones_v, deg_sh.at[idx_v.at[j]], add=True)
        return carry
    lax.fori_loop(0, NB, body, 0)

    plsc.subcore_barrier()

    @pl.when(s == 0)
    def _out():
        pltpu.sync_copy(deg_sh, out_hbm.at[c])


def _deg_call(rowd):
    f = pl.kernel(
        _deg_body,
        out_type=jax.ShapeDtypeStruct((NC, DEGP), jnp.float32),
        mesh=_mesh(),
        scratch_types=[
            pltpu.VMEM((NB, G), jnp.int32),
            pltpu.VMEM((G,), jnp.float32),
            pltpu.VMEM((2048,), jnp.float32),
            pltpu.VMEM_SHARED((DEGP,), jnp.float32),
        ],
    )
    return f(rowd)


def _pos_body(col_ref, l128_ref, s80_ref, one_ref, pos_ref, colloc_ref):
    w = pl.program_id(0)
    colv = col_ref[0]
    ism = colv < NH
    m = ism.astype(jnp.float32)
    m1 = 1.0 - m
    a0 = jnp.dot(m, l128_ref[...], preferred_element_type=jnp.float32)
    a1 = jnp.dot(m1, l128_ref[...], preferred_element_type=jnp.float32)
    rs0 = jnp.dot(m, one_ref[...], preferred_element_type=jnp.float32)
    rs1 = jnp.dot(m1, one_ref[...], preferred_element_type=jnp.float32)
    b0 = jnp.dot(s80_ref[...], rs0, preferred_element_type=jnp.float32)
    b1 = jnp.dot(s80_ref[...], rs1, preferred_element_type=jnp.float32)
    base0 = (w * CAP - 1).astype(jnp.float32)
    base1 = (NW * CAP + w * CAP - 1).astype(jnp.float32)
    p0 = a0 + b0 + base0
    p1 = a1 + b1 + base1
    pos_ref[0] = jnp.where(ism, p0, p1).astype(jnp.int32)
    colloc_ref[0] = jnp.where(ism, colv, colv - NH)


def _pos_call(colpart):
    ar = jnp.arange(G, dtype=jnp.int32)
    l128 = (ar[:, None] <= ar[None, :]).astype(jnp.float32)
    ag = jnp.arange(NB, dtype=jnp.int32)
    s80 = (ag[:, None] > ag[None, :]).astype(jnp.float32)
    one = jnp.ones((G, 1), jnp.float32)
    return pl.pallas_call(
        _pos_body,
        grid=(NW,),
        in_specs=[
            pl.BlockSpec((1, NB, G), lambda w: (w, 0, 0)),
            pl.BlockSpec((G, G), lambda w: (0, 0)),
            pl.BlockSpec((NB, NB), lambda w: (0, 0)),
            pl.BlockSpec((G, 1), lambda w: (0, 0)),
        ],
        out_specs=[
            pl.BlockSpec((1, NB, G), lambda w: (w, 0, 0)),
            pl.BlockSpec((1, NB, G), lambda w: (w, 0, 0)),
        ],
        out_shape=[
            jax.ShapeDtypeStruct((NW, NB, G), jnp.int32),
            jax.ShapeDtypeStruct((NW, NB, G), jnp.int32),
        ],
    )(colpart, l128, s80, one)


def _part_body(row_hbm, colloc_hbm, pos_hbm, rowp_hbm, colp_hbm,
               row_v, col_v, pos_v, pad_v, rowl_sh, coll_sh):
    c = lax.axis_index("c")
    s = lax.axis_index("s")
    w = c * NS + s

    def fillz(i, carry):
        pad_v[pl.ds(i * 16, 16)] = jnp.zeros((16,), jnp.int32)
        return carry
    lax.fori_loop(0, 128, fillz, 0)
    for h in range(2):
        base = h * NW * CAP + w * CAP
        for off, sz in ((0, 2048), (2048, 2048), (4096, CAP - 4096)):
            pltpu.sync_copy(pad_v.at[pl.ds(0, sz)],
                            rowl_sh.at[pl.ds(base + off, sz)])

    def filld(i, carry):
        pad_v[pl.ds(i * 16, 16)] = jnp.full((16,), NH, jnp.int32)
        return carry
    lax.fori_loop(0, 128, filld, 0)
    for h in range(2):
        base = h * NW * CAP + w * CAP
        for off, sz in ((0, 2048), (2048, 2048), (4096, CAP - 4096)):
            pltpu.sync_copy(pad_v.at[pl.ds(0, sz)],
                            coll_sh.at[pl.ds(base + off, sz)])

    pltpu.sync_copy(row_hbm.at[w], row_v)
    pltpu.sync_copy(colloc_hbm.at[w], col_v)
    pltpu.sync_copy(pos_hbm.at[w], pos_v)

    def body(j, carry):
        pltpu.sync_copy(row_v.at[j], rowl_sh.at[pos_v.at[j]])
        pltpu.sync_copy(col_v.at[j], coll_sh.at[pos_v.at[j]])
        return carry
    lax.fori_loop(0, NB, body, 0)

    for h in range(2):
        base = h * NW * CAP + w * CAP
        pltpu.sync_copy(rowl_sh.at[pl.ds(base, CAP)],
                        rowp_hbm.at[pl.ds(base, CAP)])
        pltpu.sync_copy(coll_sh.at[pl.ds(base, CAP)],
                        colp_hbm.at[pl.ds(base, CAP)])


def _part_call(rowpart, colloc, pos):
    f = pl.kernel(
        _part_body,
        out_type=(jax.ShapeDtypeStruct((NC * NW * CAP,), jnp.int32),
                  jax.ShapeDtypeStruct((NC * NW * CAP,), jnp.int32)),
        mesh=_mesh(),
        scratch_types=[
            pltpu.VMEM((NB, G), jnp.int32),
            pltpu.VMEM((NB, G), jnp.int32),
            pltpu.VMEM((NB, G), jnp.int32),
            pltpu.VMEM((2048,), jnp.int32),
            pltpu.VMEM_SHARED((NC * NW * CAP,), jnp.int32),
            pltpu.VMEM_SHARED((NC * NW * CAP,), jnp.int32),
        ],
    )
    return f(rowpart, colloc, pos)


def _spmv_body(y_hbm, rowg_hbm, colg_hbm, out_hbm,
               idxr_v, idxc_v, rows_v, y_sh, agg_sh, semg):
    c = lax.axis_index("c")
    s = lax.axis_index("s")

    pltpu.sync_copy(y_hbm.at[pl.ds(s * YRW, YRW)], y_sh.at[pl.ds(s * YRW, YRW)])

    @pl.when(s == NS - 1)
    def _ytail():
        pltpu.sync_copy(y_hbm.at[pl.ds(NS * YRW, N - NS * YRW)],
                        y_sh.at[pl.ds(NS * YRW, N - NS * YRW)])

    def zfill(i, carry):
        for jj in range(8):
            rows_v[i, pl.ds(jj * 16, 16)] = jnp.zeros((16,), jnp.float32)
        return carry
    lax.fori_loop(0, G2, zfill, 0)
    for t in range(TRW // G2):
        pltpu.sync_copy(rows_v, agg_sh.at[pl.ds(s * TRW + t * G2, G2)])
    pltpu.sync_copy(rows_v.at[pl.ds(0, TRW % G2)],
                    agg_sh.at[pl.ds(s * TRW + (TRW // G2) * G2, TRW % G2)])

    @pl.when(s == NS - 1)
    def _ztail():
        pltpu.sync_copy(rows_v.at[pl.ds(0, AGH - NS * TRW)],
                        agg_sh.at[pl.ds(NS * TRW, AGH - NS * TRW)])

    plsc.subcore_barrier()

    def chunk(ch, carry):
        pltpu.sync_copy(rowg_hbm.at[c, s, pl.ds(ch * CH, CH)], idxr_v)
        pltpu.sync_copy(colg_hbm.at[c, s, pl.ds(ch * CH, CH)], idxc_v)
        for b in range(CH):
            pltpu.async_copy(y_sh.at[idxr_v.at[b]], rows_v, semg).wait()
            pltpu.sync_copy(rows_v, agg_sh.at[idxc_v.at[b]], add=True)
        return carry
    lax.fori_loop(0, NCHK2, chunk, 0)

    plsc.subcore_barrier()
    pltpu.sync_copy(agg_sh.at[pl.ds(s * TRW, TRW)],
                    out_hbm.at[pl.ds(c * NH + s * TRW, TRW)])

    @pl.when(s == NS - 1)
    def _tail():
        pltpu.sync_copy(agg_sh.at[pl.ds(NS * TRW, NH - NS * TRW)],
                        out_hbm.at[pl.ds(c * NH + NS * TRW, NH - NS * TRW)])


def _spmv_call(y, rowg, colg):
    f = pl.kernel(
        _spmv_body,
        out_type=jax.ShapeDtypeStruct((N, D), jnp.float32),
        mesh=_mesh(),
        scratch_types=[
            pltpu.VMEM((CH, G2), jnp.int32),
            pltpu.VMEM((CH, G2), jnp.int32),
            pltpu.VMEM((G2, D), jnp.float32),
            pltpu.VMEM_SHARED((N, D), jnp.float32),
            pltpu.VMEM_SHARED((AGH, D), jnp.float32),
            pltpu.SemaphoreType.DMA,
        ],
    )
    return f(y, rowg, colg)


def _prep_body(degp_ref, x_ref, dis_ref, diag_ref, y_ref):
    deg = degp_ref[0] + degp_ref[1]
    pos = deg > 0.0
    dis = jnp.where(pos, lax.rsqrt(jnp.where(pos, deg, 1.0)), 0.0)
    dis_ref[...] = dis
    diag_ref[...] = jnp.where(pos, 0.0, -1.0)
    y_ref[...] = dis * x_ref[...]


def _prep_call(degp, x):
    return pl.pallas_call(
        _prep_body,
        grid=(NBLK,),
        in_specs=[
            pl.BlockSpec((NC, BN, 1), lambda i: (0, i, 0)),
            pl.BlockSpec((BN, D), lambda i: (i, 0)),
        ],
        out_specs=[
            pl.BlockSpec((BN, 1), lambda i: (i, 0)),
            pl.BlockSpec((BN, 1), lambda i: (i, 0)),
            pl.BlockSpec((BN, D), lambda i: (i, 0)),
        ],
        out_shape=[
            jax.ShapeDtypeStruct((N, 1), jnp.float32),
            jax.ShapeDtypeStruct((N, 1), jnp.float32),
            jax.ShapeDtypeStruct((N, D), jnp.float32),
        ],
    )(degp, x)


def _update_body(agg_ref, xc_ref, xo_ref, dis_ref, diag_ref, tx_ref, y_ref,
                 *, alpha, beta):
    dis = dis_ref[...]
    lap = diag_ref[...] * xc_ref[...] - dis * agg_ref[...]
    txn = alpha * lap - beta * xo_ref[...]
    tx_ref[...] = txn
    y_ref[...] = dis * txn


def _update_call(agg, xc, xo, dis, diag, alpha, beta):
    return pl.pallas_call(
        functools.partial(_update_body, alpha=alpha, beta=beta),
        grid=(NBLK,),
        in_specs=[
            pl.BlockSpec((BN, D), lambda i: (i, 0)),
            pl.BlockSpec((BN, D), lambda i: (i, 0)),
            pl.BlockSpec((BN, D), lambda i: (i, 0)),
            pl.BlockSpec((BN, 1), lambda i: (i, 0)),
            pl.BlockSpec((BN, 1), lambda i: (i, 0)),
        ],
        out_specs=[
            pl.BlockSpec((BN, D), lambda i: (i, 0)),
            pl.BlockSpec((BN, D), lambda i: (i, 0)),
        ],
        out_shape=[
            jax.ShapeDtypeStruct((N, D), jnp.float32),
            jax.ShapeDtypeStruct((N, D), jnp.float32),
        ],
    )(agg, xc, xo, dis, diag)


def _mm_body(tx_ref, w_ref, b_ref, out_ref, acc_ref):
    k = pl.program_id(1)

    @pl.when(k == 0)
    def _init():
        acc_ref[...] = jnp.zeros_like(acc_ref)

    acc_ref[...] += jnp.dot(tx_ref[0], w_ref[0],
                            preferred_element_type=jnp.float32)

    @pl.when(k == K - 1)
    def _fin():
        out_ref[...] = jnp.maximum(acc_ref[...] + b_ref[...], 0.0)


def _mm_call(txstack, W, b2):
    return pl.pallas_call(
        _mm_body,
        grid=(NBLK, K),
        in_specs=[
            pl.BlockSpec((1, BN, D), lambda i, k: (k, i, 0)),
            pl.BlockSpec((1, D, D), lambda i, k: (k, 0, 0)),
            pl.BlockSpec((1, D), lambda i, k: (0, 0)),
        ],
        out_specs=pl.BlockSpec((BN, D), lambda i, k: (i, 0)),
        out_shape=jax.ShapeDtypeStruct((N, D), jnp.float32),
        scratch_shapes=[pltpu.VMEM((BN, D), jnp.float32)],
        compiler_params=pltpu.CompilerParams(
            dimension_semantics=("parallel", "arbitrary")),
    )(txstack, W, b2)


def kernel(node_emb, edge_index, W, b):
    row = edge_index[0]
    col = edge_index[1]
    row_w = row.reshape(NW, EW)
    col_w = col.reshape(NW, EW)
    rowpart = jnp.pad(row_w, ((0, 0), (0, EWP - EW))).reshape(NW, NB, G)
    colpart = jnp.pad(col_w, ((0, 0), (0, EWP - EW)),
                      constant_values=N).reshape(NW, NB, G)
    rowd = jnp.pad(row_w, ((0, 0), (0, EWP - EW)),
                   constant_values=DUMP).reshape(NW, NB, G)

    pos, colloc = _pos_call(colpart)
    rowp, colp = _part_call(rowpart, colloc, pos)
    rowg = rowp.reshape(NC, NS, NBT2, G2)
    colg = colp.reshape(NC, NS, NBT2, G2)

    degp = _deg_call(rowd)
    dis, diag, y = _prep_call(degp[:, :N].reshape(NC, N, 1), node_emb)

    txs = [node_emb]
    agg = _spmv_call(y, rowg, colg)
    tx, y = _update_call(agg, node_emb, node_emb, dis, diag, 1.0, 0.0)
    txs.append(tx)
    xo, xc = node_emb, tx
    for _ in range(2, K):
        agg = _spmv_call(y, rowg, colg)
        tx, y = _update_call(agg, xc, xo, dis, diag, 2.0, 1.0)
        txs.append(tx)
        xo, xc = xc, tx

    txstack = jnp.stack(txs, axis=0)
    return _mm_call(txstack, W, b.reshape(1, D))
